# baseline (speedup 1.0000x reference)
import functools

import jax
import jax.numpy as jnp
from jax import lax
from jax.experimental import pallas as pl
from jax.experimental.pallas import tpu as pltpu
from jax.experimental.pallas import tpu_sc as plsc

_NUM_EMB = 1000000
_EMB_DIM = 64
_BATCH = 16384
_HIST = 20
_QMAX_F = 65535.0
_EPS = 0.0001 / 65535
_MAGIC = 12582912.0

_B = _BATCH * _HIST
_NW = 32
_BPW = _B // _NW
_C = 512
_IDX_MINOR = 128
_C_ROWS = _C // _IDX_MINOR
_SUPER = 1024
_NSUPER = _BPW // _SUPER

_MM_BLOCK = 5000
_MM_GRID = (_NUM_EMB * _EMB_DIM) // (_MM_BLOCK * 128)


def _qparams_body(w_ref, scale_ref, zp_ref, acc_ref):
    i = pl.program_id(0)
    bmin = jnp.min(w_ref[...])
    bmax = jnp.max(w_ref[...])

    @pl.when(i == 0)
    def _init():
        acc_ref[0] = bmin
        acc_ref[1] = bmax

    @pl.when(i > 0)
    def _acc():
        acc_ref[0] = jnp.minimum(acc_ref[0], bmin)
        acc_ref[1] = jnp.maximum(acc_ref[1], bmax)

    @pl.when(i == pl.num_programs(0) - 1)
    def _finish():
        mn = jnp.minimum(acc_ref[0], 0.0)
        mx = jnp.maximum(acc_ref[1], 0.0)
        sc = jnp.maximum((mx - mn) / _QMAX_F, jnp.float32(_EPS))
        zp = jnp.clip(-jnp.round(mn / sc), 0.0, _QMAX_F)
        scale_ref[0, 0] = sc
        zp_ref[0, 0] = zp


def _tc_qparams(weight):
    w2 = weight.reshape(-1, 128)
    scale, zp = pl.pallas_call(
        _qparams_body,
        grid=(_MM_GRID,),
        in_specs=[pl.BlockSpec((_MM_BLOCK, 128), lambda i: (i, 0))],
        out_specs=[
            pl.BlockSpec(memory_space=pltpu.SMEM),
            pl.BlockSpec(memory_space=pltpu.SMEM),
        ],
        out_shape=[jax.ShapeDtypeStruct((1, 1), jnp.float32)] * 2,
        scratch_shapes=[pltpu.SMEM((2,), jnp.float32)],
    )(w2)
    return scale[0, 0], zp[0, 0]


def _sc_gather_quant(x2d, weight, params):
    mesh = plsc.VectorSubcoreMesh(core_axis_name="c", subcore_axis_name="s")

    @functools.partial(
        pl.kernel,
        mesh=mesh,
        compiler_params=pltpu.CompilerParams(use_tc_tiling_on_sc=False),
        out_type=jax.ShapeDtypeStruct((_B, _EMB_DIM), jnp.float32),
        scratch_types=[
            pltpu.VMEM((_SUPER // _IDX_MINOR, _IDX_MINOR), jnp.int32),
            pltpu.VMEM((_C, _EMB_DIM), jnp.float32),
            pltpu.VMEM((4, 16), jnp.float32),
            pltpu.SemaphoreType.DMA,
        ],
    )
    def k(x_hbm, w_hbm, p_hbm, out_hbm, idx_v, rows_v, p_v, sem):
        wid = lax.axis_index("s") * 2 + lax.axis_index("c")
        pltpu.sync_copy(p_hbm, p_v)
        inv_scale = p_v[0, :]
        zp = p_v[1, :]
        scale = p_v[2, :]
        base0 = wid * _BPW

        def super_body(si, carry):
            base_s = pl.multiple_of(base0 + si * _SUPER, _SUPER)
            pltpu.sync_copy(
                x_hbm.at[
                    pl.ds(
                        pl.multiple_of(base_s // _IDX_MINOR, 8),
                        _SUPER // _IDX_MINOR,
                    )
                ],
                idx_v,
            )
            for h in range(_SUPER // _C):
                copies = [
                    pltpu.async_copy(
                        w_hbm.at[idx_v.at[h * _C_ROWS + j]],
                        rows_v.at[pl.ds(j * _IDX_MINOR, _IDX_MINOR)],
                        sem,
                    )
                    for j in range(_C_ROWS)
                ]
                for cp in copies:
                    cp.wait()

                def row_body(r, c2):
                    for j in range(_EMB_DIM // 16):
                        v = rows_v[r, pl.ds(j * 16, 16)]
                        t = v * inv_scale + zp
                        t = jnp.minimum(jnp.maximum(t, 0.0), _QMAX_F)
                        t = (t + _MAGIC) - _MAGIC
                        rows_v[r, pl.ds(j * 16, 16)] = (t - zp) * scale
                    return c2

                lax.fori_loop(0, _C, row_body, 0)
                pltpu.sync_copy(
                    rows_v,
                    out_hbm.at[pl.ds(pl.multiple_of(base_s + h * _C, _C), _C)],
                )
            return carry

        lax.fori_loop(0, _NSUPER, super_body, 0)

    return k(x2d, weight, params)


def kernel(x, weight):
    scale, zp = _tc_qparams(weight)
    inv_scale = 1.0 / scale
    params = jnp.stack(
        [
            jnp.full((16,), inv_scale, jnp.float32),
            jnp.full((16,), zp, jnp.float32),
            jnp.full((16,), scale, jnp.float32),
            jnp.zeros((16,), jnp.float32),
        ]
    )
    x2d = x.reshape(-1).astype(jnp.int32).reshape(-1, _IDX_MINOR)
    out = _sc_gather_quant(x2d, weight, params)
    return out.reshape(_BATCH, _HIST, _EMB_DIM)
